# baseline (speedup 1.0000x reference)
import functools

import jax
import jax.numpy as jnp
from jax import lax
from jax.experimental import pallas as pl
from jax.experimental.pallas import tpu as pltpu
from jax.experimental.pallas import tpu_sc as plsc

MAXLEN = 200
EMBED = 64
BATCH = 1024
VOCAB = 1000000

_INFO = plsc.get_sparse_core_info()
NC = _INFO.num_cores
NS = _INFO.num_subcores
NW = NC * NS
L = 16

BCOLS = 384
NBLK = (VOCAB - 64) // BCOLS
PAIRS = BCOLS // 2
TAIL0 = NBLK * BCOLS

_PARAMS = pltpu.CompilerParams(
    use_tc_tiling_on_sc=True, needs_layout_passes=False)
_MESH = dict(core_axis_name="c", subcore_axis_name="s")


def _detile_body(tblt, tailt, scratch, vb, ob, vtail, obtail,
                 insem, outsem):
    wid = lax.axis_index("s") * NC + lax.axis_index("c")
    iota = lax.iota(jnp.int32, L)

    jrow = []
    jcol = []
    for g in range(BCOLS // L):
        jv = iota + g * L
        jrow.append(jv // 2)
        jcol.append((jv % 2) * EMBED)

    nblk_w = pl.cdiv(NBLK - wid, NW)

    def transpose_block(p):
        @plsc.parallel_loop(0, EMBED, unroll=2)
        def _(d):
            for g in range(BCOLS // L):
                v = vb[p, d, pl.ds(g * L, L)]
                plsc.store_scatter(ob.at[p], [jrow[g], jcol[g] + d], v)

    def blk_body(i, carry):
        p = lax.rem(i, 2)
        cb = wid + i * NW
        @pl.when(i + 1 < nblk_w)
        def _():
            nxt = wid + (i + 1) * NW
            c0 = pl.multiple_of(nxt * BCOLS, 128)
            pltpu.async_copy(tblt.at[:, pl.ds(c0, BCOLS)],
                             vb.at[1 - p], insem.at[1 - p])
        pltpu.make_async_copy(tblt.at[:, pl.ds(0, BCOLS)],
                              vb.at[p], insem.at[p]).wait()
        @pl.when(i >= 2)
        def _():
            pltpu.make_async_copy(
                ob.at[p], scratch.at[pl.ds(0, PAIRS)], outsem.at[p]).wait()
        transpose_block(p)
        r0 = pl.multiple_of(cb * PAIRS, 8)
        pltpu.async_copy(ob.at[p], scratch.at[pl.ds(r0, PAIRS)],
                         outsem.at[p])
        return carry

    @pl.when(nblk_w > 0)
    def _():
        c0 = pl.multiple_of(wid * BCOLS, 128)
        pltpu.async_copy(tblt.at[:, pl.ds(c0, BCOLS)], vb.at[0],
                         insem.at[0])
        lax.fori_loop(0, nblk_w, blk_body, 0)
        @pl.when(nblk_w >= 2)
        def _():
            pltpu.make_async_copy(
                ob.at[lax.rem(nblk_w, 2)], scratch.at[pl.ds(0, PAIRS)],
                outsem.at[lax.rem(nblk_w, 2)]).wait()
        pltpu.make_async_copy(
            ob.at[lax.rem(nblk_w + 1, 2)], scratch.at[pl.ds(0, PAIRS)],
            outsem.at[lax.rem(nblk_w + 1, 2)]).wait()

    @pl.when(wid == NW - 1)
    def _():
        pltpu.sync_copy(tailt, vtail)

        @plsc.parallel_loop(0, EMBED, unroll=2)
        def _(d):
            for g in range(64 // L):
                jv = iota + g * L
                v = vtail[d, pl.ds(g * L, L)]
                plsc.store_scatter(obtail, [jv // 2, (jv % 2) * EMBED + d], v)
        pltpu.sync_copy(obtail, scratch.at[pl.ds(TAIL0 // 2, 32)])


NBB = BATCH // 128
NTB = NW // NBB
TROWS = MAXLEN // NTB


def _gather_body(scr, xt, posp, out_t, xv, posv, gidx, colb, posbuf,
                 gbuf, obuf, gsem, osem):
    wid = lax.axis_index("s") * NC + lax.axis_index("c")
    tb = wid // NBB
    bb = wid % NBB
    b0 = pl.multiple_of(bb * 128, 128)
    t0 = tb * TROWS
    iota = lax.iota(jnp.int32, L)

    pltpu.sync_copy(xt.at[:, pl.ds(b0, 128)], xv)
    pltpu.sync_copy(posp, posv)

    def fire_gather(i, p):
        t = t0 + i
        for g in range(8):
            sl = pl.ds(g * L, L)
            v = xv[t, sl]
            gidx[sl] = lax.mul(lax.shift_right_logical(v, 7), EMBED) + \
                lax.bitwise_and(v, EMBED - 1)
        pltpu.async_copy(scr.at[gidx], gbuf.at[p], gsem.at[p])

    def t_body(i, carry):
        p = lax.rem(i, 2)
        t = t0 + i
        pltpu.make_async_copy(scr.at[gidx], gbuf.at[p], gsem.at[p]).wait()
        cb_local = []
        for g in range(8):
            v = xv[t, pl.ds(g * L, L)]
            cb_local.append(lax.mul(
                lax.bitwise_and(lax.shift_right_logical(v, 6), 1), EMBED))
        for dd in range(EMBED // L):
            pv = posv[t, pl.ds(dd * L, L)]
            for k in range(L):
                kvec = jnp.full((L,), k, jnp.int32)
                posbuf[dd * L + k, pl.ds(0, L)] = pv.at[kvec].get(
                    mode="promise_in_bounds")
        @pl.when(i + 1 < TROWS)
        def _():
            fire_gather(i + 1, 1 - p)
        @pl.when(i >= 2)
        def _():
            pltpu.make_async_copy(
                obuf.at[p], out_t.at[t, :, pl.ds(b0, 128)], osem.at[p]).wait()
        @plsc.parallel_loop(0, EMBED, unroll=2)
        def _(d):
            pvec = posbuf[d, pl.ds(0, L)]
            for g in range(8):
                sl = pl.ds(g * L, L)
                jv = iota + g * L
                cv = cb_local[g] + d
                vals = plsc.load_gather(gbuf.at[p], [jv, cv])
                obuf[p, d, sl] = vals + pvec
        pltpu.async_copy(obuf.at[p], out_t.at[t, :, pl.ds(b0, 128)],
                         osem.at[p])
        return carry

    fire_gather(0, 0)
    lax.fori_loop(0, TROWS, t_body, 0)
    t_last = t0 + TROWS - 1
    pltpu.make_async_copy(
        obuf.at[lax.rem(TROWS, 2)],
        out_t.at[t_last, :, pl.ds(b0, 128)],
        osem.at[lax.rem(TROWS, 2)]).wait()
    pltpu.make_async_copy(
        obuf.at[lax.rem(TROWS + 1, 2)],
        out_t.at[t_last, :, pl.ds(b0, 128)],
        osem.at[lax.rem(TROWS + 1, 2)]).wait()


def _tc_transpose_body(in_ref, out_ref):
    t = in_ref[...].T
    out_ref[:, 0:EMBED] = t[0:EMBED, :]
    out_ref[:, EMBED:2 * EMBED] = t[EMBED:2 * EMBED, :]


def _tc_transpose(tblt):
    grid = (VOCAB + 127) // 128
    return pl.pallas_call(
        _tc_transpose_body,
        grid=(grid,),
        in_specs=[pl.BlockSpec((EMBED, 128), lambda i: (0, i))],
        out_specs=pl.BlockSpec((EMBED, 2 * EMBED), lambda i: (i, 0)),
        out_shape=jax.ShapeDtypeStruct((VOCAB // 2, 2 * EMBED), jnp.float32),
        compiler_params=pltpu.CompilerParams(
            dimension_semantics=("arbitrary",)),
    )(tblt)


@functools.partial(jax.jit, static_argnames=())
def kernel(x, token_table, pos_table):
    xt = x.T.astype(jnp.int32)
    posp = jnp.pad(pos_table, ((0, 0), (0, 64)))

    scratch = _tc_transpose(token_table.T)

    gather = pl.kernel(
        _gather_body,
        out_type=jax.ShapeDtypeStruct((MAXLEN, EMBED, BATCH), jnp.float32),
        mesh=plsc.VectorSubcoreMesh(**_MESH),
        compiler_params=_PARAMS,
        scratch_types=[
            pltpu.VMEM((MAXLEN, 128), jnp.int32),
            pltpu.VMEM((MAXLEN, 128), jnp.float32),
            pltpu.VMEM((128,), jnp.int32),
            pltpu.VMEM((128,), jnp.int32),
            pltpu.VMEM((EMBED, 128), jnp.float32),
            pltpu.VMEM((2, 128, 128), jnp.float32),
            pltpu.VMEM((2, EMBED, 128), jnp.float32),
            pltpu.SemaphoreType.DMA((2,)),
            pltpu.SemaphoreType.DMA((2,)),
        ],
    )
    out_t = gather(scratch, xt, posp)
    return out_t.transpose(2, 0, 1)


# speedup vs baseline: 5.5876x; 5.5876x over previous
import functools

import jax
import jax.numpy as jnp
from jax import lax
from jax.experimental import pallas as pl
from jax.experimental.pallas import tpu as pltpu
from jax.experimental.pallas import tpu_sc as plsc

MAXLEN = 200
EMBED = 64
BATCH = 1024
VOCAB = 1000000

_INFO = plsc.get_sparse_core_info()
NC = _INFO.num_cores
NS = _INFO.num_subcores
NW = NC * NS
L = 16

BCOLS = 384
NBLK = (VOCAB - 64) // BCOLS
PAIRS = BCOLS // 2
TAIL0 = NBLK * BCOLS

_PARAMS = pltpu.CompilerParams(
    use_tc_tiling_on_sc=True, needs_layout_passes=False)
_MESH = dict(core_axis_name="c", subcore_axis_name="s")


def _detile_body(tblt, tailt, scratch, vb, ob, vtail, obtail,
                 insem, outsem):
    wid = lax.axis_index("s") * NC + lax.axis_index("c")
    iota = lax.iota(jnp.int32, L)

    jrow = []
    jcol = []
    for g in range(BCOLS // L):
        jv = iota + g * L
        jrow.append(jv // 2)
        jcol.append((jv % 2) * EMBED)

    nblk_w = pl.cdiv(NBLK - wid, NW)

    def transpose_block(p):
        @plsc.parallel_loop(0, EMBED, unroll=2)
        def _(d):
            for g in range(BCOLS // L):
                v = vb[p, d, pl.ds(g * L, L)]
                plsc.store_scatter(ob.at[p], [jrow[g], jcol[g] + d], v)

    def blk_body(i, carry):
        p = lax.rem(i, 2)
        cb = wid + i * NW
        @pl.when(i + 1 < nblk_w)
        def _():
            nxt = wid + (i + 1) * NW
            c0 = pl.multiple_of(nxt * BCOLS, 128)
            pltpu.async_copy(tblt.at[:, pl.ds(c0, BCOLS)],
                             vb.at[1 - p], insem.at[1 - p])
        pltpu.make_async_copy(tblt.at[:, pl.ds(0, BCOLS)],
                              vb.at[p], insem.at[p]).wait()
        @pl.when(i >= 2)
        def _():
            pltpu.make_async_copy(
                ob.at[p], scratch.at[pl.ds(0, PAIRS)], outsem.at[p]).wait()
        transpose_block(p)
        r0 = pl.multiple_of(cb * PAIRS, 8)
        pltpu.async_copy(ob.at[p], scratch.at[pl.ds(r0, PAIRS)],
                         outsem.at[p])
        return carry

    @pl.when(nblk_w > 0)
    def _():
        c0 = pl.multiple_of(wid * BCOLS, 128)
        pltpu.async_copy(tblt.at[:, pl.ds(c0, BCOLS)], vb.at[0],
                         insem.at[0])
        lax.fori_loop(0, nblk_w, blk_body, 0)
        @pl.when(nblk_w >= 2)
        def _():
            pltpu.make_async_copy(
                ob.at[lax.rem(nblk_w, 2)], scratch.at[pl.ds(0, PAIRS)],
                outsem.at[lax.rem(nblk_w, 2)]).wait()
        pltpu.make_async_copy(
            ob.at[lax.rem(nblk_w + 1, 2)], scratch.at[pl.ds(0, PAIRS)],
            outsem.at[lax.rem(nblk_w + 1, 2)]).wait()

    @pl.when(wid == NW - 1)
    def _():
        pltpu.sync_copy(tailt, vtail)

        @plsc.parallel_loop(0, EMBED, unroll=2)
        def _(d):
            for g in range(64 // L):
                jv = iota + g * L
                v = vtail[d, pl.ds(g * L, L)]
                plsc.store_scatter(obtail, [jv // 2, (jv % 2) * EMBED + d], v)
        pltpu.sync_copy(obtail, scratch.at[pl.ds(TAIL0 // 2, 32)])


NBB = BATCH // 128
NTB = NW // NBB
TROWS = MAXLEN // NTB


def _gather_body(scr, xt, posp, out_t, xv, posv, gidx, colb, posbuf,
                 gbuf, obuf, gsem, osem):
    wid = lax.axis_index("s") * NC + lax.axis_index("c")
    tb = wid // NBB
    bb = wid % NBB
    b0 = pl.multiple_of(bb * 128, 128)
    t0 = tb * TROWS
    iota = lax.iota(jnp.int32, L)

    pltpu.sync_copy(xt.at[:, pl.ds(b0, 128)], xv)
    pltpu.sync_copy(posp, posv)

    def fire_gather(i, p):
        t = t0 + i
        for g in range(8):
            sl = pl.ds(g * L, L)
            v = xv[t, sl]
            gidx[sl] = lax.shift_right_logical(v, 1)
        pltpu.async_copy(scr.at[gidx], gbuf.at[p], gsem.at[p])

    def t_body(i, carry):
        p = lax.rem(i, 2)
        t = t0 + i
        pltpu.make_async_copy(scr.at[gidx], gbuf.at[p], gsem.at[p]).wait()
        cb_local = []
        for g in range(8):
            v = xv[t, pl.ds(g * L, L)]
            cb_local.append(lax.mul(lax.bitwise_and(v, 1), EMBED))
        for dd in range(EMBED // L):
            pv = posv[t, pl.ds(dd * L, L)]
            for k in range(L):
                kvec = jnp.full((L,), k, jnp.int32)
                posbuf[dd * L + k, pl.ds(0, L)] = pv.at[kvec].get(
                    mode="promise_in_bounds")
        @pl.when(i + 1 < TROWS)
        def _():
            fire_gather(i + 1, 1 - p)
        @pl.when(i >= 2)
        def _():
            pltpu.make_async_copy(
                obuf.at[p], out_t.at[t, :, pl.ds(b0, 128)], osem.at[p]).wait()
        @plsc.parallel_loop(0, EMBED, unroll=4)
        def _(d):
            pvec = posbuf[d, pl.ds(0, L)]
            for g in range(8):
                sl = pl.ds(g * L, L)
                jv = iota + g * L
                cv = cb_local[g] + d
                vals = plsc.load_gather(gbuf.at[p], [jv, cv])
                obuf[p, d, sl] = vals + pvec
        pltpu.async_copy(obuf.at[p], out_t.at[t, :, pl.ds(b0, 128)],
                         osem.at[p])
        return carry

    fire_gather(0, 0)
    lax.fori_loop(0, TROWS, t_body, 0)
    t_last = t0 + TROWS - 1
    pltpu.make_async_copy(
        obuf.at[lax.rem(TROWS, 2)],
        out_t.at[t_last, :, pl.ds(b0, 128)],
        osem.at[lax.rem(TROWS, 2)]).wait()
    pltpu.make_async_copy(
        obuf.at[lax.rem(TROWS + 1, 2)],
        out_t.at[t_last, :, pl.ds(b0, 128)],
        osem.at[lax.rem(TROWS + 1, 2)]).wait()


@functools.partial(jax.jit, static_argnames=())
def kernel(x, token_table, pos_table):
    xt = x.T.astype(jnp.int32)
    posp = jnp.pad(pos_table, ((0, 0), (0, 64)))

    scratch = token_table.reshape(VOCAB // 2, 2 * EMBED)

    gather = pl.kernel(
        _gather_body,
        out_type=jax.ShapeDtypeStruct((MAXLEN, EMBED, BATCH), jnp.float32),
        mesh=plsc.VectorSubcoreMesh(**_MESH),
        compiler_params=_PARAMS,
        scratch_types=[
            pltpu.VMEM((MAXLEN, 128), jnp.int32),
            pltpu.VMEM((MAXLEN, 128), jnp.float32),
            pltpu.VMEM((128,), jnp.int32),
            pltpu.VMEM((128,), jnp.int32),
            pltpu.VMEM((EMBED, 128), jnp.float32),
            pltpu.VMEM((2, 128, 128), jnp.float32),
            pltpu.VMEM((2, EMBED, 128), jnp.float32),
            pltpu.SemaphoreType.DMA((2,)),
            pltpu.SemaphoreType.DMA((2,)),
        ],
    )
    out_t = gather(scratch, xt, posp)
    return out_t.transpose(2, 0, 1)
